# baseline (speedup 1.0000x reference)
import functools

import jax
import jax.numpy as jnp
from jax.experimental import pallas as pl
from jax.experimental.pallas import tpu as pltpu

_IN, _H, _OUT = 2, 32, 2
_W1_R, _B1_R, _W2_R, _B2_R, _W3_R, _B3_R = 0, 8, 16, 48, 56, 88
_C = 8
_GROUP_ROWS = 2 * _C
_UNIT_ROWS = 2 * _GROUP_ROWS


def _mlp_kernel(x_ref, w1_ref, w2_ref, w3_ref, b2_ref, b3_ref,
                o_ref, *, bm, unroll):
    w1 = w1_ref[...]
    w2 = w2_ref[...]
    w3 = w3_ref[...]
    b2 = b2_ref[...]
    b3 = b3_ref[...]

    def unit(base, mega):
        slices = []
        for m in range(2 * mega):
            slices.append(x_ref[pl.ds(base + _GROUP_ROWS * m, _GROUP_ROWS), :])
        xg = jnp.concatenate(
            [jnp.concatenate(slices, axis=1),
             jnp.ones((1, 128 * 2 * mega), jnp.float32)], axis=0)
        s1 = jax.lax.dot_general(
            xg, w1, (((0,), (0,)), ((), ())),
            preferred_element_type=jnp.float32)
        h1 = jnp.maximum(s1, 0.0)
        h2 = jnp.maximum(
            jnp.dot(h1, w2, preferred_element_type=jnp.float32) + b2, 0.0)
        u3 = jnp.dot(h2, w3, preferred_element_type=jnp.float32)
        u = u3.T + b3[:, :128 * 2 * mega]
        for m in range(2 * mega):
            o_ref[pl.ds(base + _GROUP_ROWS * m, _GROUP_ROWS), :] = (
                u[:, 128 * m:128 * (m + 1)])

    mega, nun = unroll
    rows_per_iter = _UNIT_ROWS * mega * nun
    n_loop, rem_rows = divmod(bm, rows_per_iter)

    if n_loop > 0:
        def body(i, carry):
            for k in range(nun):
                unit(i * rows_per_iter + k * _UNIT_ROWS * mega, mega)
            return carry
        jax.lax.fori_loop(0, n_loop, body, 0, unroll=False)
    base = n_loop * rows_per_iter
    while rem_rows > 0:
        unit(base, 1)
        base += _UNIT_ROWS
        rem_rows -= _UNIT_ROWS


def _pack_operands(packed_params):
    p = packed_params.astype(jnp.float32)
    w1 = p[_W1_R:_W1_R + _IN, :]
    b1 = p[_B1_R, :]
    w2 = p[_W2_R:_W2_R + _H, :]
    b2 = p[_B2_R, :]
    w3 = p[_W3_R:_W3_R + _H, :_OUT]
    b3 = p[_B3_R, :_OUT]

    eye = jnp.eye(_C, dtype=jnp.float32)
    w1bd = jnp.concatenate(
        [jnp.kron(eye, w1), jnp.tile(b1, _C)[None, :]], axis=0)
    w2bd = jnp.kron(eye, w2)
    w3bd = jnp.kron(eye, w3)
    b2r = jnp.tile(b2, _C)[None, :]
    return w1bd, w2bd, w3bd, b2r, jnp.tile(b3, _C)


def kernel(x, packed_params):
    B = x.shape[0]
    assert B % 128 == 0, "batch must be a multiple of 128"
    R = B // 64
    xv = x.reshape(B // 128, 128, _IN).transpose(0, 2, 1).reshape(R, 128)

    w1bd, w2bd, w3bd, b2r, b3t = _pack_operands(packed_params)

    Rp = -(-R // _UNIT_ROWS) * _UNIT_ROWS
    if Rp != R:
        xv = jnp.pad(xv, ((0, Rp - R), (0, 0)))

    bm = Rp
    for cand in (8192, 4096, 2048, 1024, 512, 256, 128, 64, 32):
        if Rp % cand == 0:
            bm = cand
            break
    grid = (Rp // bm,)
    unroll = next(((m, n) for (m, n) in ((64, 2), (32, 2), (16, 2), (16, 1), (8, 2), (4, 2), (4, 1), (2, 1), (1, 1))
                   if (bm // _UNIT_ROWS) % (m * n) == 0), (1, 1))

    b3m = jnp.broadcast_to(b3t[:, None], (_C * _OUT, 256 * unroll[0]))
    kern = functools.partial(_mlp_kernel, bm=bm, unroll=unroll)
    flops = 2 * B * (_IN * _H + _H * _H + _H * _OUT)
    yv = pl.pallas_call(
        kern,
        out_shape=jax.ShapeDtypeStruct((Rp, 128), jnp.float32),
        grid=grid,
        in_specs=[
            pl.BlockSpec((bm, 128), lambda i: (i, 0)),
            pl.BlockSpec((2 * _C + 1, 256), lambda i: (0, 0)),
            pl.BlockSpec((_C * _H, 256), lambda i: (0, 0)),
            pl.BlockSpec((_C * _H, 2 * _C), lambda i: (0, 0)),
            pl.BlockSpec((1, 256), lambda i: (0, 0)),
            pl.BlockSpec((_C * _OUT, 256 * unroll[0]), lambda i: (0, 0)),
        ],
        out_specs=pl.BlockSpec((bm, 128), lambda i: (i, 0)),
        compiler_params=pltpu.CompilerParams(
            dimension_semantics=("parallel",)),
        cost_estimate=pl.CostEstimate(
            flops=flops, transcendentals=0,
            bytes_accessed=8 * B * _IN + 4 * 96 * 32),
    )(xv, w1bd, w2bd, w3bd, b2r, b3m)

    if Rp != R:
        yv = yv[:R]
    return yv.reshape(B // 128, _IN, 128).transpose(0, 2, 1).reshape(B, _IN)


# speedup vs baseline: 1.0160x; 1.0160x over previous
import functools

import jax
import jax.numpy as jnp
from jax.experimental import pallas as pl
from jax.experimental.pallas import tpu as pltpu

_IN, _H, _OUT = 2, 32, 2
_W1_R, _B1_R, _W2_R, _B2_R, _W3_R, _B3_R = 0, 8, 16, 48, 56, 88
_C = 8
_GROUP_ROWS = 2 * _C
_UNIT_ROWS = 2 * _GROUP_ROWS


def _mlp_kernel(x_ref, w1_ref, w2_ref, w3_ref, b2_ref, b3_ref,
                o_ref, *, bm, unroll):
    w1 = w1_ref[...]
    w2 = w2_ref[...]
    w3 = w3_ref[...]
    b2 = b2_ref[...]
    b3 = b3_ref[...]

    def unit(base, mega):
        slices = []
        for m in range(2 * mega):
            slices.append(x_ref[pl.ds(base + _GROUP_ROWS * m, _GROUP_ROWS), :])
        xg = jnp.concatenate(
            [jnp.concatenate(slices, axis=1),
             jnp.ones((1, 128 * 2 * mega), jnp.float32)], axis=0)
        s1 = jax.lax.dot_general(
            xg, w1, (((0,), (0,)), ((), ())),
            preferred_element_type=jnp.float32)
        h1 = jnp.maximum(s1, 0.0)
        h2 = jnp.maximum(
            jnp.dot(h1, w2, preferred_element_type=jnp.float32) + b2, 0.0)
        u3 = jnp.dot(h2, w3, preferred_element_type=jnp.float32)
        u = u3.T + b3[:, :128 * 2 * mega]
        for m in range(2 * mega):
            o_ref[pl.ds(base + _GROUP_ROWS * m, _GROUP_ROWS), :] = (
                u[:, 128 * m:128 * (m + 1)])

    mega, nun = unroll
    rows_per_iter = _UNIT_ROWS * mega * nun
    n_loop, rem_rows = divmod(bm, rows_per_iter)

    if n_loop > 0:
        def body(i, carry):
            for k in range(nun):
                unit(i * rows_per_iter + k * _UNIT_ROWS * mega, mega)
            return carry
        jax.lax.fori_loop(0, n_loop, body, 0, unroll=False)
    base = n_loop * rows_per_iter
    while rem_rows > 0:
        unit(base, 1)
        base += _UNIT_ROWS
        rem_rows -= _UNIT_ROWS


def _pack_operands(packed_params):
    p = packed_params.astype(jnp.float32)
    w1 = p[_W1_R:_W1_R + _IN, :]
    b1 = p[_B1_R, :]
    w2 = p[_W2_R:_W2_R + _H, :]
    b2 = p[_B2_R, :]
    w3 = p[_W3_R:_W3_R + _H, :_OUT]
    b3 = p[_B3_R, :_OUT]

    eye = jnp.eye(_C, dtype=jnp.float32)
    w1bd = jnp.concatenate(
        [jnp.kron(eye, w1), jnp.tile(b1, _C)[None, :]], axis=0)
    w2bd = jnp.kron(eye, w2)
    w3bd = jnp.kron(eye, w3)
    b2r = jnp.tile(b2, _C)[None, :]
    return w1bd, w2bd, w3bd, b2r, jnp.tile(b3, _C)


def kernel(x, packed_params):
    B = x.shape[0]
    assert B % 128 == 0, "batch must be a multiple of 128"
    R = B // 64
    xv = x.reshape(B // 128, 128, _IN).transpose(0, 2, 1).reshape(R, 128)

    w1bd, w2bd, w3bd, b2r, b3t = _pack_operands(packed_params)

    Rp = -(-R // _UNIT_ROWS) * _UNIT_ROWS
    if Rp != R:
        xv = jnp.pad(xv, ((0, Rp - R), (0, 0)))

    bm = Rp
    for cand in (4096, 2048, 1024, 512, 256, 128, 64, 32):
        if Rp % cand == 0:
            bm = cand
            break
    grid = (Rp // bm,)
    unroll = next(((m, n) for (m, n) in ((32, 4), (64, 2), (32, 2), (16, 2), (16, 1), (8, 2), (4, 2), (4, 1), (2, 1), (1, 1))
                   if (bm // _UNIT_ROWS) % (m * n) == 0), (1, 1))

    b3m = jnp.broadcast_to(b3t[:, None], (_C * _OUT, 256 * unroll[0]))
    kern = functools.partial(_mlp_kernel, bm=bm, unroll=unroll)
    flops = 2 * B * (_IN * _H + _H * _H + _H * _OUT)
    yv = pl.pallas_call(
        kern,
        out_shape=jax.ShapeDtypeStruct((Rp, 128), jnp.float32),
        grid=grid,
        in_specs=[
            pl.BlockSpec((bm, 128), lambda i: (i, 0)),
            pl.BlockSpec((2 * _C + 1, 256), lambda i: (0, 0)),
            pl.BlockSpec((_C * _H, 256), lambda i: (0, 0)),
            pl.BlockSpec((_C * _H, 2 * _C), lambda i: (0, 0)),
            pl.BlockSpec((1, 256), lambda i: (0, 0)),
            pl.BlockSpec((_C * _OUT, 256 * unroll[0]), lambda i: (0, 0)),
        ],
        out_specs=pl.BlockSpec((bm, 128), lambda i: (i, 0)),
        compiler_params=pltpu.CompilerParams(
            dimension_semantics=("parallel",)),
        cost_estimate=pl.CostEstimate(
            flops=flops, transcendentals=0,
            bytes_accessed=8 * B * _IN + 4 * 96 * 32),
    )(xv, w1bd, w2bd, w3bd, b2r, b3m)

    if Rp != R:
        yv = yv[:R]
    return yv.reshape(B // 128, _IN, 128).transpose(0, 2, 1).reshape(B, _IN)
